# initial kernel scaffold (unmeasured)
import jax
import jax.numpy as jnp
from jax import lax
from jax.experimental import pallas as pl
from jax.experimental.pallas import tpu as pltpu

N_DEV = 4
SQ = 2048
HC = 1024
NH = 8
DH = 128
NG = 4
GS = SQ // NG
SCALE = 0.08838834764831843


def _group_rows(a):
    s = a.shape[0]
    return a.reshape(8, 4, 64, -1).transpose(1, 0, 2, 3).reshape(s, -1)


def _ungroup_rows(a):
    s = a.shape[0]
    return a.reshape(4, 8, 64, -1).transpose(1, 0, 2, 3).reshape(s, -1)


def _body(x_ref, wq_ref, wo_ref, k_ref, v_ref, out_ref,
          comm_ref, ctx_ref, send_sems, recv_sems):
    my = lax.axis_index("i")
    left = lax.rem(my + N_DEV - 1, N_DEV)
    right = lax.rem(my + 1, N_DEV)

    barrier_sem = pltpu.get_barrier_semaphore()
    for nbr in (left, right):
        pl.semaphore_signal(barrier_sem, inc=1, device_id=(nbr,),
                            device_id_type=pl.DeviceIdType.MESH)
    pl.semaphore_wait(barrier_sem, 2)

    comm_ref[0, 0] = wq_ref[...]
    comm_ref[0, 1] = wo_ref[...]

    xv = x_ref[...]

    for h in range(N_DEV):
        if h < N_DEV - 1:
            rdma = pltpu.make_async_remote_copy(
                src_ref=comm_ref.at[h],
                dst_ref=comm_ref.at[h + 1],
                send_sem=send_sems.at[h],
                recv_sem=recv_sems.at[h + 1],
                device_id=(right,),
                device_id_type=pl.DeviceIdType.MESH,
            )
            rdma.start()

        jj = lax.rem(my + N_DEV - h, N_DEV)
        kc = k_ref[jj]
        vc = v_ref[jj]

        qv = (jnp.dot(xv, comm_ref[h, 0],
                      preferred_element_type=jnp.float32)
              * SCALE).astype(jnp.bfloat16)

        for g in range(NG):
            for hh in range(NH):
                q1 = qv[g * GS:(g + 1) * GS, hh * DH:(hh + 1) * DH]
                k1 = kc[g * GS:(g + 1) * GS, hh * DH:(hh + 1) * DH]
                v1 = vc[g * GS:(g + 1) * GS, hh * DH:(hh + 1) * DH]
                s = lax.dot_general(q1, k1, (((1,), (1,)), ((), ())),
                                    preferred_element_type=jnp.float32)
                m = jnp.max(s, axis=1, keepdims=True)
                p = jnp.exp(s - m)
                p = (p / jnp.sum(p, axis=1, keepdims=True)).astype(jnp.bfloat16)
                ctx1 = jnp.dot(p, v1, preferred_element_type=jnp.float32)
                ctx_ref[g * GS:(g + 1) * GS, hh * DH:(hh + 1) * DH] = (
                    ctx1.astype(jnp.bfloat16))

        partial = jnp.dot(ctx_ref[...], comm_ref[h, 1],
                          preferred_element_type=jnp.float32)
        if h == 0:
            out_ref[...] = partial
        else:
            out_ref[...] += partial

        if h < N_DEV - 1:
            rdma.wait()


def kernel(x, Wq, K_ext, V_ext, Wo):
    my = lax.axis_index("i")
    xg = _group_rows(x[0].astype(jnp.bfloat16))
    wq = Wq.astype(jnp.bfloat16)
    wo = Wo.astype(jnp.bfloat16)
    kb = K_ext[my].astype(jnp.bfloat16).reshape(SQ, NG, HC)
    vb = V_ext[my].astype(jnp.bfloat16).reshape(SQ, NG, HC)
    kb = _group_rows(kb.transpose(1, 0, 2).reshape(NG * SQ, HC)
                     .reshape(NG, SQ, HC).reshape(NG * SQ, HC))
    kb = kb.reshape(NG, SQ, HC)
    vb = vb.transpose(1, 0, 2)
    vb = jnp.stack([_group_rows(vb[c]) for c in range(NG)], axis=0)

    out = pl.pallas_call(
        _body,
        out_shape=jax.ShapeDtypeStruct((SQ, 1024), jnp.float32),
        in_specs=[pl.BlockSpec(memory_space=pltpu.VMEM)] * 5,
        out_specs=pl.BlockSpec(memory_space=pltpu.VMEM),
        scratch_shapes=[
            pltpu.VMEM((N_DEV, 2, 1024, HC), jnp.bfloat16),
            pltpu.VMEM((SQ, HC), jnp.bfloat16),
            pltpu.SemaphoreType.DMA((N_DEV,)),
            pltpu.SemaphoreType.DMA((N_DEV,)),
        ],
        compiler_params=pltpu.CompilerParams(collective_id=0),
    )(xg, wq, wo, kb, vb)

    return _ungroup_rows(out)[None].astype(jnp.float32)


# baseline (device time: 341312 ns/iter reference)
import jax
import jax.numpy as jnp
from jax import lax
from jax.experimental import pallas as pl
from jax.experimental.pallas import tpu as pltpu

N_DEV = 4
SQ = 2048
HC = 1024
NH = 8
DH = 128
NG = 4
GS = SQ // NG
SCALE = 0.08838834764831843


def _group_rows(a):
    s = a.shape[0]
    return a.reshape(8, 4, 64, -1).transpose(1, 0, 2, 3).reshape(s, -1)


def _ungroup_rows(a):
    s = a.shape[0]
    return a.reshape(4, 8, 64, -1).transpose(1, 0, 2, 3).reshape(s, -1)


def _body(x_ref, wq_ref, wo_ref, k_hbm, v_hbm, out_ref,
          comm_ref, kbuf, vbuf, send_sems, recv_sems, copy_sems):
    my = lax.axis_index("i")
    left = lax.rem(my + N_DEV - 1, N_DEV)
    right = lax.rem(my + 1, N_DEV)

    barrier_sem = pltpu.get_barrier_semaphore()
    for nbr in (left, right):
        pl.semaphore_signal(barrier_sem, inc=1, device_id=(nbr,),
                            device_id_type=pl.DeviceIdType.MESH)
    pl.semaphore_wait(barrier_sem, 2)

    comm_ref[0, 0] = wq_ref[...]
    comm_ref[0, 1] = wo_ref[...]

    for h in range(N_DEV):
        if h < N_DEV - 1:
            rdma = pltpu.make_async_remote_copy(
                src_ref=comm_ref.at[h],
                dst_ref=comm_ref.at[h + 1],
                send_sem=send_sems.at[h],
                recv_sem=recv_sems.at[h + 1],
                device_id=(right,),
                device_id_type=pl.DeviceIdType.MESH,
            )
            rdma.start()

        jj = lax.rem(my + N_DEV - h, N_DEV)
        kcp = pltpu.make_async_copy(k_hbm.at[jj], kbuf, copy_sems.at[0])
        vcp = pltpu.make_async_copy(v_hbm.at[jj], vbuf, copy_sems.at[1])
        kcp.start()
        vcp.start()
        kcp.wait()
        vcp.wait()

        for g in range(NG):
            qg = (jnp.dot(x_ref[g * GS:(g + 1) * GS, :], comm_ref[h, 0],
                          preferred_element_type=jnp.float32)
                  * SCALE).astype(jnp.bfloat16)
            ctx_heads = []
            for hh in range(NH):
                q1 = qg[:, hh * DH:(hh + 1) * DH]
                k1 = kbuf[g * GS:(g + 1) * GS, hh * DH:(hh + 1) * DH]
                v1 = vbuf[g * GS:(g + 1) * GS, hh * DH:(hh + 1) * DH]
                s = lax.dot_general(q1, k1, (((1,), (1,)), ((), ())),
                                    preferred_element_type=jnp.float32)
                m = jnp.max(s, axis=1, keepdims=True)
                p = jnp.exp(s - m)
                p = (p / jnp.sum(p, axis=1, keepdims=True)).astype(jnp.bfloat16)
                ctx_heads.append(
                    jnp.dot(p, v1, preferred_element_type=jnp.float32)
                    .astype(jnp.bfloat16))
            ctx_g = jnp.concatenate(ctx_heads, axis=1)
            partial = jnp.dot(ctx_g, comm_ref[h, 1],
                              preferred_element_type=jnp.float32)
            if h == 0:
                out_ref[g * GS:(g + 1) * GS, :] = partial
            else:
                out_ref[g * GS:(g + 1) * GS, :] += partial

        if h < N_DEV - 1:
            rdma.wait()


def kernel(x, Wq, K_ext, V_ext, Wo):
    my = lax.axis_index("i")
    xg = _group_rows(x[0].astype(jnp.bfloat16))
    wq = Wq.astype(jnp.bfloat16)
    wo = Wo.astype(jnp.bfloat16)
    kb = K_ext[my].astype(jnp.bfloat16).reshape(SQ, NG, HC).transpose(1, 0, 2)
    vb = V_ext[my].astype(jnp.bfloat16).reshape(SQ, NG, HC).transpose(1, 0, 2)
    kb = jnp.stack([_group_rows(kb[c]) for c in range(NG)], axis=0)
    vb = jnp.stack([_group_rows(vb[c]) for c in range(NG)], axis=0)

    out = pl.pallas_call(
        _body,
        out_shape=jax.ShapeDtypeStruct((SQ, 1024), jnp.float32),
        in_specs=[pl.BlockSpec(memory_space=pltpu.VMEM)] * 3
        + [pl.BlockSpec(memory_space=pl.ANY)] * 2,
        out_specs=pl.BlockSpec(memory_space=pltpu.VMEM),
        scratch_shapes=[
            pltpu.VMEM((N_DEV, 2, 1024, HC), jnp.bfloat16),
            pltpu.VMEM((SQ, HC), jnp.bfloat16),
            pltpu.VMEM((SQ, HC), jnp.bfloat16),
            pltpu.SemaphoreType.DMA((N_DEV,)),
            pltpu.SemaphoreType.DMA((N_DEV,)),
            pltpu.SemaphoreType.DMA((2,)),
        ],
        compiler_params=pltpu.CompilerParams(
            collective_id=0, vmem_limit_bytes=100 * 1024 * 1024),
    )(xg, wq, wo, kb, vb)

    return _ungroup_rows(out)[None].astype(jnp.float32)


# device time: 201783 ns/iter; 1.6915x vs baseline; 1.6915x over previous
import jax
import jax.numpy as jnp
from jax import lax
from jax.experimental import pallas as pl
from jax.experimental.pallas import tpu as pltpu

N_DEV = 4
SQ = 2048
NB = 32
BS = 64
HC = 1024
NH = 8
DH = 128
NG = 4
GS = SQ // NG
SCALE = 0.08838834764831843


def _group_rows(a):
    s = a.shape[0]
    return a.reshape(8, 4, 64, -1).transpose(1, 0, 2, 3).reshape(s, -1)


def _ungroup_rows(a):
    s = a.shape[0]
    return a.reshape(4, 8, 64, -1).transpose(1, 0, 2, 3).reshape(s, -1)


def _body(x_ref, wq_ref, wo_ref, k_hbm, v_hbm, out_ref,
          wq_comm, wo_comm, kbuf, vbuf, ctx_hold,
          wq_ssem, wq_rsem, wo_ssem, wo_rsem, copy_sems):
    my = lax.axis_index("i")
    left = lax.rem(my + N_DEV - 1, N_DEV)
    right = lax.rem(my + 1, N_DEV)

    barrier_sem = pltpu.get_barrier_semaphore()
    for nbr in (left, right):
        pl.semaphore_signal(barrier_sem, inc=1, device_id=(nbr,),
                            device_id_type=pl.DeviceIdType.MESH)
    pl.semaphore_wait(barrier_sem, 2)

    wq_comm[0] = wq_ref[...]
    wo_comm[0] = wo_ref[...]

    for h in range(N_DEV):
        if h < N_DEV - 1:
            rq = pltpu.make_async_remote_copy(
                src_ref=wq_comm.at[h], dst_ref=wq_comm.at[h + 1],
                send_sem=wq_ssem.at[h], recv_sem=wq_rsem.at[h + 1],
                device_id=(right,), device_id_type=pl.DeviceIdType.MESH)
            ro = pltpu.make_async_remote_copy(
                src_ref=wo_comm.at[h], dst_ref=wo_comm.at[h + 1],
                send_sem=wo_ssem.at[h], recv_sem=wo_rsem.at[h + 1],
                device_id=(left,), device_id_type=pl.DeviceIdType.MESH)
            rq.start()
            ro.start()

        aj = lax.rem(my + N_DEV - h, N_DEV)
        cps = []
        for g in range(NG):
            for k in range(NG * 2):
                b = NG * k + g
                cols = pl.ds(aj * HC, HC)
                cps.append(pltpu.make_async_copy(
                    k_hbm.at[b, :, cols], kbuf.at[g, pl.ds(BS * k, BS)],
                    copy_sems.at[0]))
                cps.append(pltpu.make_async_copy(
                    v_hbm.at[b, :, cols], vbuf.at[g, pl.ds(BS * k, BS)],
                    copy_sems.at[1]))
        for c in cps:
            c.start()
        for c in cps:
            c.wait()

        for g in range(NG):
            qg = jnp.dot(x_ref[g * GS:(g + 1) * GS, :], wq_comm[h],
                         preferred_element_type=jnp.float32
                         ).astype(jnp.bfloat16)
            ctx_heads = []
            for hh in range(NH):
                q1 = qg[:, hh * DH:(hh + 1) * DH]
                k1 = kbuf[g, :, hh * DH:(hh + 1) * DH]
                v1 = vbuf[g, :, hh * DH:(hh + 1) * DH]
                s = lax.dot_general(q1, k1, (((1,), (1,)), ((), ())),
                                    preferred_element_type=jnp.float32)
                m = jnp.max(s, axis=1, keepdims=True)
                p = jnp.exp(s - m)
                p = (p / jnp.sum(p, axis=1, keepdims=True)).astype(jnp.bfloat16)
                ctx_heads.append(
                    jnp.dot(p, v1, preferred_element_type=jnp.float32)
                    .astype(jnp.bfloat16))
            ctx_g = jnp.concatenate(ctx_heads, axis=1)
            rows = pl.ds(g * GS, GS)
            if h == 0:
                out_ref[rows, :] = jnp.dot(
                    ctx_g, wo_comm[0], preferred_element_type=jnp.float32)
            elif h == 1:
                ctx_hold[rows, :] = ctx_g
            elif h == 2:
                out_ref[rows, :] += jnp.dot(
                    ctx_g, wo_comm[2], preferred_element_type=jnp.float32)
            else:
                out_ref[rows, :] += jnp.dot(
                    ctx_g, wo_comm[1], preferred_element_type=jnp.float32)
                out_ref[rows, :] += jnp.dot(
                    ctx_hold[rows, :], wo_comm[3],
                    preferred_element_type=jnp.float32)

        if h < N_DEV - 1:
            rq.wait()
            ro.wait()


def kernel(x, Wq, K_ext, V_ext, Wo):
    my = lax.axis_index("i")
    xg = _group_rows((x[0] * SCALE).astype(jnp.bfloat16))
    wq = Wq.astype(jnp.bfloat16)
    wo = Wo.astype(jnp.bfloat16)
    kb = K_ext[my].astype(jnp.bfloat16).reshape(NB, BS, NG * HC)
    vb = V_ext[my].astype(jnp.bfloat16).reshape(NB, BS, NG * HC)

    out = pl.pallas_call(
        _body,
        out_shape=jax.ShapeDtypeStruct((SQ, 1024), jnp.float32),
        in_specs=[pl.BlockSpec(memory_space=pltpu.VMEM)] * 3
        + [pl.BlockSpec(memory_space=pl.ANY)] * 2,
        out_specs=pl.BlockSpec(memory_space=pltpu.VMEM),
        scratch_shapes=[
            pltpu.VMEM((N_DEV, 1024, HC), jnp.bfloat16),
            pltpu.VMEM((N_DEV, 1024, HC), jnp.bfloat16),
            pltpu.VMEM((NG, GS, HC), jnp.bfloat16),
            pltpu.VMEM((NG, GS, HC), jnp.bfloat16),
            pltpu.VMEM((SQ, HC), jnp.bfloat16),
            pltpu.SemaphoreType.DMA((N_DEV,)),
            pltpu.SemaphoreType.DMA((N_DEV,)),
            pltpu.SemaphoreType.DMA((N_DEV,)),
            pltpu.SemaphoreType.DMA((N_DEV,)),
            pltpu.SemaphoreType.DMA((2,)),
        ],
        compiler_params=pltpu.CompilerParams(
            collective_id=0, vmem_limit_bytes=100 * 1024 * 1024),
    )(xg, wq, wo, kb, vb)

    return _ungroup_rows(out)[None].astype(jnp.float32)
